# baseline (speedup 1.0000x reference)
import functools

import jax
import jax.numpy as jnp
from jax import lax
from jax.experimental import pallas as pl
from jax.experimental.pallas import tpu as pltpu
from jax.experimental.pallas import tpu_sc as plsc

N_NODES = 10000
N_EDGES = 320000
NP = 10240
EP = 327680
EC = 128
NROWS = EP // EC
NC, NS = 2, 16
NW = NC * NS
CPW = NROWS // NW
RPT = NP // NS

D_IN = 128
D_HID = 64
D_O = 16


def _sc_mesh():
    return plsc.VectorSubcoreMesh(core_axis_name="c", subcore_axis_name="s")


@functools.partial(
    pl.kernel,
    out_type=jax.ShapeDtypeStruct((NC, NP, 8), jnp.float32),
    mesh=_sc_mesh(),
    scratch_types=[
        pltpu.VMEM_SHARED((NP, 8), jnp.float32),
        pltpu.VMEM((CPW, EC), jnp.int32),
        pltpu.VMEM((EC, 8), jnp.float32),
    ],
    compiler_params=pltpu.CompilerParams(use_tc_tiling_on_sc=False),
    name="deg_hist",
)
def _deg_kernel(dst2d, zeros_hbm, ones_hbm, out, acc, idx_v, ones_v):
    c = lax.axis_index("c")
    s = lax.axis_index("s")
    wid = s * NC + c
    pltpu.sync_copy(ones_hbm, ones_v)
    pltpu.sync_copy(dst2d.at[pl.ds(wid * CPW, CPW)], idx_v)
    pltpu.sync_copy(zeros_hbm, acc.at[pl.ds(s * RPT, RPT)])
    plsc.subcore_barrier()

    def body(j, carry):
        pltpu.sync_copy(ones_v, acc.at[idx_v.at[j]], add=True)
        return carry

    lax.fori_loop(0, CPW, body, 0)
    plsc.subcore_barrier()
    pltpu.sync_copy(acc.at[pl.ds(s * RPT, RPT)], out.at[c, pl.ds(s * RPT, RPT)])


def _make_spmm(d):
    nbuf = 2 if d == D_HID else 4
    @functools.partial(
        pl.kernel,
        out_type=jax.ShapeDtypeStruct((NC, NP, d), jnp.float32),
        mesh=_sc_mesh(),
        scratch_types=[
            pltpu.VMEM_SHARED((NP, d), jnp.float32),
            pltpu.VMEM_SHARED((NP, d), jnp.float32),
            pltpu.VMEM((CPW, EC), jnp.int32),
            pltpu.VMEM((CPW, EC), jnp.int32),
            [pltpu.VMEM((EC, d), jnp.float32)] * nbuf,
            [pltpu.SemaphoreType.DMA] * nbuf,
        ],
        compiler_params=pltpu.CompilerParams(use_tc_tiling_on_sc=False),
        name=f"spmm{d}",
    )
    def spmm(table, src2d, dst2d, zeros_hbm, out, acc, table_s,
             src_v, dst_v, rows, gsem):
        c = lax.axis_index("c")
        s = lax.axis_index("s")
        wid = s * NC + c
        pltpu.sync_copy(src2d.at[pl.ds(wid * CPW, CPW)], src_v)
        pltpu.sync_copy(dst2d.at[pl.ds(wid * CPW, CPW)], dst_v)
        pltpu.sync_copy(zeros_hbm, acc.at[pl.ds(s * RPT, RPT)])
        pltpu.sync_copy(table.at[pl.ds(s * RPT, RPT)], table_s.at[pl.ds(s * RPT, RPT)])
        plsc.subcore_barrier()

        def body(j, carry):
            pltpu.sync_copy(table_s.at[src_v.at[j]], rows[0])
            pltpu.sync_copy(rows[0], acc.at[dst_v.at[j]], add=True)
            return carry

        lax.fori_loop(0, CPW, body, 0)
        plsc.subcore_barrier()
        pltpu.sync_copy(acc.at[pl.ds(s * RPT, RPT)], out.at[c, pl.ds(s * RPT, RPT)])

    return spmm


_spmm64 = _make_spmm(D_HID)
_spmm16 = _make_spmm(D_O)


_BN = 512
_GRID = NP // _BN


def _dis(d0_ref, d1_ref):
    deg = d0_ref[:] + d1_ref[:] + 1.0
    return lax.rsqrt(deg)


def _tc1_body(x_ref, w_ref, d0_ref, d1_ref, o_ref):
    dis = _dis(d0_ref, d1_ref)
    h = jnp.dot(x_ref[:], w_ref[:], preferred_element_type=jnp.float32)
    o_ref[:] = h * dis[:, None]


def _tc2_body(a0_ref, a1_ref, h1_ref, d0_ref, d1_ref, w2_ref, b1_ref, o_ref):
    dis = _dis(d0_ref, d1_ref)
    z = dis[:, None] * (a0_ref[:] + a1_ref[:] + h1_ref[:]) + b1_ref[:]
    z = jnp.maximum(z, 0.0)
    h2 = jnp.dot(z, w2_ref[:], preferred_element_type=jnp.float32)
    o_ref[:] = h2 * dis[:, None]


def _tc3_body(a0_ref, a1_ref, h2_ref, d0_ref, d1_ref, b2_ref, o_ref):
    dis = _dis(d0_ref, d1_ref)
    o_ref[:] = dis[:, None] * (a0_ref[:] + a1_ref[:] + h2_ref[:]) + b2_ref[:]


def _row_spec(d):
    return pl.BlockSpec((_BN, d), lambda i: (i, 0))


def _vec_spec():
    return pl.BlockSpec((_BN,), lambda i: (i,))


def _full_spec(shape):
    return pl.BlockSpec(shape, lambda i: tuple(0 for _ in shape))


def _tc1(xp, W1, d0, d1):
    return pl.pallas_call(
        _tc1_body,
        grid=(_GRID,),
        in_specs=[_row_spec(D_IN), _full_spec((D_IN, D_HID)), _vec_spec(), _vec_spec()],
        out_specs=_row_spec(D_HID),
        out_shape=jax.ShapeDtypeStruct((NP, D_HID), jnp.float32),
    )(xp, W1, d0, d1)


def _tc2(a0, a1, h1p, d0, d1, W2p, b1):
    return pl.pallas_call(
        _tc2_body,
        grid=(_GRID,),
        in_specs=[
            _row_spec(D_HID), _row_spec(D_HID), _row_spec(D_HID),
            _vec_spec(), _vec_spec(),
            _full_spec((D_HID, D_O)), _full_spec((1, D_HID)),
        ],
        out_specs=_row_spec(D_O),
        out_shape=jax.ShapeDtypeStruct((NP, D_O), jnp.float32),
    )(a0, a1, h1p, d0, d1, W2p, b1)


def _tc3(a0, a1, h2p, d0, d1, b2p):
    return pl.pallas_call(
        _tc3_body,
        grid=(_GRID,),
        in_specs=[
            _row_spec(D_O), _row_spec(D_O), _row_spec(D_O),
            _vec_spec(), _vec_spec(),
            _full_spec((1, D_O)),
        ],
        out_specs=_row_spec(D_O),
        out_shape=jax.ShapeDtypeStruct((NP, D_O), jnp.float32),
    )(a0, a1, h2p, d0, d1, b2p)


def kernel(x, edge_index, W1, b1, W2, b2):
    src = jnp.asarray(edge_index[0], jnp.int32)
    dst = jnp.asarray(edge_index[1], jnp.int32)
    src2d = jnp.pad(src, (0, EP - N_EDGES)).reshape(NROWS, EC)
    dst2d = jnp.pad(dst, (0, EP - N_EDGES), constant_values=N_NODES).reshape(NROWS, EC)
    xp = jnp.pad(x, ((0, NP - N_NODES), (0, 0)))
    W2p = jnp.pad(W2, ((0, 0), (0, D_O - W2.shape[1])))
    b2p = jnp.pad(b2, (0, D_O - b2.shape[0])).reshape(1, D_O)
    b1r = b1.reshape(1, D_HID)

    z8 = jnp.zeros((RPT, 8), jnp.float32)
    z64 = jnp.zeros((RPT, D_HID), jnp.float32)
    z16 = jnp.zeros((RPT, D_O), jnp.float32)
    ones8 = jnp.ones((EC, 8), jnp.float32)

    deg_pair = _deg_kernel(dst2d, z8, ones8)
    d0 = deg_pair[0, :, 0]
    d1 = deg_pair[1, :, 0]

    h1p = _tc1(xp, W1, d0, d1)
    acc1 = _spmm64(h1p, src2d, dst2d, z64)
    h2p = _tc2(acc1[0], acc1[1], h1p, d0, d1, W2p, b1r)
    acc2 = _spmm16(h2p, src2d, dst2d, z16)
    outp = _tc3(acc2[0], acc2[1], h2p, d0, d1, b2p)
    return outp[:N_NODES, :2]


# speedup vs baseline: 1.1516x; 1.1516x over previous
import functools

import jax
import jax.numpy as jnp
from jax import lax
from jax.experimental import pallas as pl
from jax.experimental.pallas import tpu as pltpu
from jax.experimental.pallas import tpu_sc as plsc

N_NODES = 10000
N_EDGES = 320000
NP = 10240
EP = 327680
EC = 128
NROWS = EP // EC
NC, NS = 2, 16
NW = NC * NS
CPW = NROWS // NW
RPT = NP // NS

D_IN = 128
D_HID = 64
D_O = 16


def _sc_mesh():
    return plsc.VectorSubcoreMesh(core_axis_name="c", subcore_axis_name="s")


@functools.partial(
    pl.kernel,
    out_type=jax.ShapeDtypeStruct((NC, NP, 8), jnp.float32),
    mesh=_sc_mesh(),
    scratch_types=[
        pltpu.VMEM_SHARED((NP, 8), jnp.float32),
        pltpu.VMEM((CPW, EC), jnp.int32),
        pltpu.VMEM((EC, 8), jnp.float32),
    ],
    compiler_params=pltpu.CompilerParams(use_tc_tiling_on_sc=False),
    name="deg_hist",
)
def _deg_kernel(dst2d, zeros_hbm, ones_hbm, out, acc, idx_v, ones_v):
    c = lax.axis_index("c")
    s = lax.axis_index("s")
    wid = s * NC + c
    pltpu.sync_copy(ones_hbm, ones_v)
    pltpu.sync_copy(dst2d.at[pl.ds(wid * CPW, CPW)], idx_v)
    pltpu.sync_copy(zeros_hbm, acc.at[pl.ds(s * RPT, RPT)])
    plsc.subcore_barrier()

    def body(j, carry):
        pltpu.sync_copy(ones_v, acc.at[idx_v.at[j]], add=True)
        return carry

    lax.fori_loop(0, CPW, body, 0)
    plsc.subcore_barrier()
    pltpu.sync_copy(acc.at[pl.ds(s * RPT, RPT)], out.at[c, pl.ds(s * RPT, RPT)])


def _make_spmm(d):
    nbuf = 2 if d == D_HID else 4
    @functools.partial(
        pl.kernel,
        out_type=jax.ShapeDtypeStruct((NC, NP, d), jnp.float32),
        mesh=_sc_mesh(),
        scratch_types=[
            pltpu.VMEM_SHARED((NP, d), jnp.float32),
            pltpu.VMEM_SHARED((NP, d), jnp.float32),
            pltpu.VMEM((CPW, EC), jnp.int32),
            pltpu.VMEM((CPW, EC), jnp.int32),
            [pltpu.VMEM((EC, d), jnp.float32)] * nbuf,
            [pltpu.SemaphoreType.DMA] * nbuf,
        ],
        compiler_params=pltpu.CompilerParams(use_tc_tiling_on_sc=False),
        name=f"spmm{d}",
    )
    def spmm(table, src2d, dst2d, zeros_hbm, out, acc, table_s,
             src_v, dst_v, rows, gsem):
        c = lax.axis_index("c")
        s = lax.axis_index("s")
        wid = s * NC + c
        pltpu.sync_copy(src2d.at[pl.ds(wid * CPW, CPW)], src_v)
        pltpu.sync_copy(dst2d.at[pl.ds(wid * CPW, CPW)], dst_v)
        pltpu.sync_copy(zeros_hbm, acc.at[pl.ds(s * RPT, RPT)])
        pltpu.sync_copy(table.at[pl.ds(s * RPT, RPT)], table_s.at[pl.ds(s * RPT, RPT)])
        plsc.subcore_barrier()

        pltpu.sync_copy(table_s.at[src_v.at[0]], rows[0])
        pltpu.async_copy(rows[0], acc.at[dst_v.at[0]], gsem[0], add=True)
        pltpu.sync_copy(table_s.at[src_v.at[1]], rows[1])
        pltpu.async_copy(rows[1], acc.at[dst_v.at[1]], gsem[1], add=True)

        def body(k, carry):
            for b in range(2):
                j = 2 * k + b
                pltpu.make_async_copy(rows[b], acc.at[dst_v.at[j - 2]], gsem[b]).wait()
                pltpu.sync_copy(table_s.at[src_v.at[j]], rows[b])
                pltpu.async_copy(rows[b], acc.at[dst_v.at[j]], gsem[b], add=True)
            return carry

        lax.fori_loop(1, CPW // 2, body, 0)
        for b in range(2):
            pltpu.make_async_copy(rows[b], acc.at[dst_v.at[CPW - 2 + b]], gsem[b]).wait()
        plsc.subcore_barrier()
        pltpu.sync_copy(acc.at[pl.ds(s * RPT, RPT)], out.at[c, pl.ds(s * RPT, RPT)])

    return spmm


_spmm64 = _make_spmm(D_HID)
_spmm16 = _make_spmm(D_O)


_BN = 512
_GRID = NP // _BN


def _dis(d0_ref, d1_ref):
    deg = d0_ref[:] + d1_ref[:] + 1.0
    return lax.rsqrt(deg)


def _tc1_body(x_ref, w_ref, d0_ref, d1_ref, o_ref):
    dis = _dis(d0_ref, d1_ref)
    h = jnp.dot(x_ref[:], w_ref[:], preferred_element_type=jnp.float32)
    o_ref[:] = h * dis[:, None]


def _tc2_body(a0_ref, a1_ref, h1_ref, d0_ref, d1_ref, w2_ref, b1_ref, o_ref):
    dis = _dis(d0_ref, d1_ref)
    z = dis[:, None] * (a0_ref[:] + a1_ref[:] + h1_ref[:]) + b1_ref[:]
    z = jnp.maximum(z, 0.0)
    h2 = jnp.dot(z, w2_ref[:], preferred_element_type=jnp.float32)
    o_ref[:] = h2 * dis[:, None]


def _tc3_body(a0_ref, a1_ref, h2_ref, d0_ref, d1_ref, b2_ref, o_ref):
    dis = _dis(d0_ref, d1_ref)
    o_ref[:] = dis[:, None] * (a0_ref[:] + a1_ref[:] + h2_ref[:]) + b2_ref[:]


def _row_spec(d):
    return pl.BlockSpec((_BN, d), lambda i: (i, 0))


def _vec_spec():
    return pl.BlockSpec((_BN,), lambda i: (i,))


def _full_spec(shape):
    return pl.BlockSpec(shape, lambda i: tuple(0 for _ in shape))


def _tc1(xp, W1, d0, d1):
    return pl.pallas_call(
        _tc1_body,
        grid=(_GRID,),
        in_specs=[_row_spec(D_IN), _full_spec((D_IN, D_HID)), _vec_spec(), _vec_spec()],
        out_specs=_row_spec(D_HID),
        out_shape=jax.ShapeDtypeStruct((NP, D_HID), jnp.float32),
    )(xp, W1, d0, d1)


def _tc2(a0, a1, h1p, d0, d1, W2p, b1):
    return pl.pallas_call(
        _tc2_body,
        grid=(_GRID,),
        in_specs=[
            _row_spec(D_HID), _row_spec(D_HID), _row_spec(D_HID),
            _vec_spec(), _vec_spec(),
            _full_spec((D_HID, D_O)), _full_spec((1, D_HID)),
        ],
        out_specs=_row_spec(D_O),
        out_shape=jax.ShapeDtypeStruct((NP, D_O), jnp.float32),
    )(a0, a1, h1p, d0, d1, W2p, b1)


def _tc3(a0, a1, h2p, d0, d1, b2p):
    return pl.pallas_call(
        _tc3_body,
        grid=(_GRID,),
        in_specs=[
            _row_spec(D_O), _row_spec(D_O), _row_spec(D_O),
            _vec_spec(), _vec_spec(),
            _full_spec((1, D_O)),
        ],
        out_specs=_row_spec(D_O),
        out_shape=jax.ShapeDtypeStruct((NP, D_O), jnp.float32),
    )(a0, a1, h2p, d0, d1, b2p)


def kernel(x, edge_index, W1, b1, W2, b2):
    src = jnp.asarray(edge_index[0], jnp.int32)
    dst = jnp.asarray(edge_index[1], jnp.int32)
    src2d = jnp.pad(src, (0, EP - N_EDGES)).reshape(NROWS, EC)
    dst2d = jnp.pad(dst, (0, EP - N_EDGES), constant_values=N_NODES).reshape(NROWS, EC)
    xp = jnp.pad(x, ((0, NP - N_NODES), (0, 0)))
    W2p = jnp.pad(W2, ((0, 0), (0, D_O - W2.shape[1])))
    b2p = jnp.pad(b2, (0, D_O - b2.shape[0])).reshape(1, D_O)
    b1r = b1.reshape(1, D_HID)

    z8 = jnp.zeros((RPT, 8), jnp.float32)
    z64 = jnp.zeros((RPT, D_HID), jnp.float32)
    z16 = jnp.zeros((RPT, D_O), jnp.float32)
    ones8 = jnp.ones((EC, 8), jnp.float32)

    deg_pair = _deg_kernel(dst2d, z8, ones8)
    d0 = deg_pair[0, :, 0]
    d1 = deg_pair[1, :, 0]

    h1p = _tc1(xp, W1, d0, d1)
    acc1 = _spmm64(h1p, src2d, dst2d, z64)
    h2p = _tc2(acc1[0], acc1[1], h1p, d0, d1, W2p, b1r)
    acc2 = _spmm16(h2p, src2d, dst2d, z16)
    outp = _tc3(acc2[0], acc2[1], h2p, d0, d1, b2p)
    return outp[:N_NODES, :2]


# speedup vs baseline: 1.3281x; 1.1533x over previous
import functools

import jax
import jax.numpy as jnp
from jax import lax
from jax.experimental import pallas as pl
from jax.experimental.pallas import tpu as pltpu
from jax.experimental.pallas import tpu_sc as plsc

N_NODES = 10000
N_EDGES = 320000
NP = 10240
EP = 327680
EC = 128
NROWS = EP // EC
NC, NS = 2, 16
NW = NC * NS
CPW = NROWS // NW
RPT = NP // NS

D_IN = 128
D_HID = 64
D_O = 16


def _sc_mesh():
    return plsc.VectorSubcoreMesh(core_axis_name="c", subcore_axis_name="s")


@functools.partial(
    pl.kernel,
    out_type=jax.ShapeDtypeStruct((NC, NP, 8), jnp.float32),
    mesh=_sc_mesh(),
    scratch_types=[
        pltpu.VMEM_SHARED((NP, 8), jnp.float32),
        pltpu.VMEM((CPW, EC), jnp.int32),
        pltpu.VMEM((EC, 8), jnp.float32),
    ],
    compiler_params=pltpu.CompilerParams(use_tc_tiling_on_sc=False),
    name="deg_hist",
)
def _deg_kernel(dst2d, zeros_hbm, ones_hbm, out, acc, idx_v, ones_v):
    c = lax.axis_index("c")
    s = lax.axis_index("s")
    wid = s * NC + c
    pltpu.sync_copy(ones_hbm, ones_v)
    pltpu.sync_copy(dst2d.at[pl.ds(wid * CPW, CPW)], idx_v)
    pltpu.sync_copy(zeros_hbm, acc.at[pl.ds(s * RPT, RPT)])
    plsc.subcore_barrier()

    def body(j, carry):
        pltpu.sync_copy(ones_v, acc.at[idx_v.at[j]], add=True)
        return carry

    lax.fori_loop(0, CPW, body, 0)
    plsc.subcore_barrier()
    pltpu.sync_copy(acc.at[pl.ds(s * RPT, RPT)], out.at[c, pl.ds(s * RPT, RPT)])


def _make_spmm(d):
    nbuf = 2 if d == D_HID else 4
    @functools.partial(
        pl.kernel,
        out_type=jax.ShapeDtypeStruct((NC, NP, d), jnp.float32),
        mesh=_sc_mesh(),
        scratch_types=[
            pltpu.VMEM_SHARED((NP, d), jnp.float32),
            pltpu.VMEM_SHARED((NP, d), jnp.float32),
            pltpu.VMEM((CPW, EC), jnp.int32),
            pltpu.VMEM((CPW, EC), jnp.int32),
            [pltpu.VMEM((EC, d), jnp.float32)] * nbuf,
            [pltpu.SemaphoreType.DMA] * nbuf,
        ],
        compiler_params=pltpu.CompilerParams(use_tc_tiling_on_sc=False),
        name=f"spmm{d}",
    )
    def spmm(table, src2d, dst2d, zeros_hbm, out, acc, table_s,
             src_v, dst_v, rows, gsem):
        c = lax.axis_index("c")
        s = lax.axis_index("s")
        wid = s * NC + c
        pltpu.sync_copy(src2d.at[pl.ds(wid * CPW, CPW)], src_v)
        pltpu.sync_copy(dst2d.at[pl.ds(wid * CPW, CPW)], dst_v)
        pltpu.sync_copy(zeros_hbm, acc.at[pl.ds(s * RPT, RPT)])
        pltpu.sync_copy(table.at[pl.ds(s * RPT, RPT)], table_s.at[pl.ds(s * RPT, RPT)])
        plsc.subcore_barrier()

        pltpu.sync_copy(table_s.at[src_v.at[0]], rows[0])
        pltpu.async_copy(rows[0], acc.at[dst_v.at[0]], gsem[0], add=True)
        pltpu.sync_copy(table_s.at[src_v.at[1]], rows[1])
        pltpu.async_copy(rows[1], acc.at[dst_v.at[1]], gsem[1], add=True)

        def body(k, carry):
            for b in range(2):
                j = 2 * k + b
                pltpu.make_async_copy(rows[b], acc.at[dst_v.at[j - 2]], gsem[b]).wait()
                pltpu.sync_copy(table_s.at[src_v.at[j]], rows[b])
                pltpu.async_copy(rows[b], acc.at[dst_v.at[j]], gsem[b], add=True)
            return carry

        lax.fori_loop(1, CPW // 2, body, 0)
        for b in range(2):
            pltpu.make_async_copy(rows[b], acc.at[dst_v.at[CPW - 2 + b]], gsem[b]).wait()
        plsc.subcore_barrier()
        pltpu.sync_copy(acc.at[pl.ds(s * RPT, RPT)], out.at[c, pl.ds(s * RPT, RPT)])

    return spmm


_spmm64 = _make_spmm(D_HID)
_spmm16 = _make_spmm(D_O)


_BN = 2048
_GRID = NP // _BN


def _dis(deg_ref):
    deg = deg_ref[0, :, 0] + deg_ref[1, :, 0] + 1.0
    return lax.rsqrt(deg)


def _tc1_body(x_ref, w_ref, deg_ref, o_ref):
    dis = _dis(deg_ref)
    h = jnp.dot(x_ref[:], w_ref[:], preferred_element_type=jnp.float32)
    o_ref[:] = h * dis[:, None]


def _tc2_body(a_ref, h1_ref, deg_ref, w2_ref, b1_ref, o_ref):
    dis = _dis(deg_ref)
    z = dis[:, None] * (a_ref[0] + a_ref[1] + h1_ref[:]) + b1_ref[:]
    z = jnp.maximum(z, 0.0)
    h2 = jnp.dot(z, w2_ref[:], preferred_element_type=jnp.float32)
    o_ref[:] = h2 * dis[:, None]


def _tc3_body(a_ref, h2_ref, deg_ref, b2_ref, o_ref):
    dis = _dis(deg_ref)
    o_ref[:] = dis[:, None] * (a_ref[0] + a_ref[1] + h2_ref[:]) + b2_ref[:]


def _row_spec(d):
    return pl.BlockSpec((_BN, d), lambda i: (i, 0))


def _pair_spec(d):
    return pl.BlockSpec((NC, _BN, d), lambda i: (0, i, 0))


def _full_spec(shape):
    return pl.BlockSpec(shape, lambda i: tuple(0 for _ in shape))


def _tc1(xp, W1, deg_pair):
    return pl.pallas_call(
        _tc1_body,
        grid=(_GRID,),
        in_specs=[_row_spec(D_IN), _full_spec((D_IN, D_HID)), _pair_spec(8)],
        out_specs=_row_spec(D_HID),
        out_shape=jax.ShapeDtypeStruct((NP, D_HID), jnp.float32),
    )(xp, W1, deg_pair)


def _tc2(acc1, h1p, deg_pair, W2p, b1):
    return pl.pallas_call(
        _tc2_body,
        grid=(_GRID,),
        in_specs=[
            _pair_spec(D_HID), _row_spec(D_HID), _pair_spec(8),
            _full_spec((D_HID, D_O)), _full_spec((1, D_HID)),
        ],
        out_specs=_row_spec(D_O),
        out_shape=jax.ShapeDtypeStruct((NP, D_O), jnp.float32),
    )(acc1, h1p, deg_pair, W2p, b1)


def _tc3(acc2, h2p, deg_pair, b2p):
    return pl.pallas_call(
        _tc3_body,
        grid=(_GRID,),
        in_specs=[
            _pair_spec(D_O), _row_spec(D_O), _pair_spec(8),
            _full_spec((1, D_O)),
        ],
        out_specs=_row_spec(D_O),
        out_shape=jax.ShapeDtypeStruct((NP, D_O), jnp.float32),
    )(acc2, h2p, deg_pair, b2p)


def kernel(x, edge_index, W1, b1, W2, b2):
    src = jnp.asarray(edge_index[0], jnp.int32)
    dst = jnp.asarray(edge_index[1], jnp.int32)
    src2d = jnp.pad(src, (0, EP - N_EDGES)).reshape(NROWS, EC)
    dst2d = jnp.pad(dst, (0, EP - N_EDGES), constant_values=N_NODES).reshape(NROWS, EC)
    xp = jnp.pad(x, ((0, NP - N_NODES), (0, 0)))
    W2p = jnp.pad(W2, ((0, 0), (0, D_O - W2.shape[1])))
    b2p = jnp.pad(b2, (0, D_O - b2.shape[0])).reshape(1, D_O)
    b1r = b1.reshape(1, D_HID)

    z8 = jnp.zeros((RPT, 8), jnp.float32)
    z64 = jnp.zeros((RPT, D_HID), jnp.float32)
    z16 = jnp.zeros((RPT, D_O), jnp.float32)
    ones8 = jnp.ones((EC, 8), jnp.float32)

    deg_pair = _deg_kernel(dst2d, z8, ones8)
    h1p = _tc1(xp, W1, deg_pair)
    acc1 = _spmm64(h1p, src2d, dst2d, z64)
    h2p = _tc2(acc1, h1p, deg_pair, W2p, b1r)
    acc2 = _spmm16(h2p, src2d, dst2d, z16)
    outp = _tc3(acc2, h2p, deg_pair, b2p)
    return outp[:N_NODES, :2]


# speedup vs baseline: 1.3296x; 1.0011x over previous
import functools

import jax
import jax.numpy as jnp
import numpy as np
from jax import lax
from jax.experimental import pallas as pl
from jax.experimental.pallas import tpu as pltpu
from jax.experimental.pallas import tpu_sc as plsc

N_NODES = 10000
N_EDGES = 320000
NP = 10240
EP = 327680
EC = 128
NROWS = EP // EC
NC, NS = 2, 16
NW = NC * NS
CPW = NROWS // NW
RPT = NP // NS

D_IN = 128
D_HID = 64
D_O = 16

_SRC_PAD = np.zeros((EP - N_EDGES,), np.int32)
_DST_PAD = np.full((EP - N_EDGES,), N_NODES, np.int32)


def _sc_mesh():
    return plsc.VectorSubcoreMesh(core_axis_name="c", subcore_axis_name="s")


@functools.partial(
    pl.kernel,
    out_type=jax.ShapeDtypeStruct((NC, NP, 8), jnp.float32),
    mesh=_sc_mesh(),
    scratch_types=[
        pltpu.VMEM_SHARED((NP, 8), jnp.float32),
        pltpu.VMEM((CPW, EC), jnp.int32),
        pltpu.VMEM((EC, 8), jnp.float32),
    ],
    compiler_params=pltpu.CompilerParams(use_tc_tiling_on_sc=False),
    name="deg_hist",
)
def _deg_kernel(dst2d, zeros_hbm, ones_hbm, out, acc, idx_v, ones_v):
    c = lax.axis_index("c")
    s = lax.axis_index("s")
    wid = s * NC + c
    pltpu.sync_copy(ones_hbm, ones_v)
    pltpu.sync_copy(dst2d.at[pl.ds(wid * CPW, CPW)], idx_v)
    pltpu.sync_copy(zeros_hbm, acc.at[pl.ds(s * RPT, RPT)])
    plsc.subcore_barrier()

    def body(j, carry):
        pltpu.sync_copy(ones_v, acc.at[idx_v.at[j]], add=True)
        return carry

    lax.fori_loop(0, CPW, body, 0)
    plsc.subcore_barrier()
    pltpu.sync_copy(acc.at[pl.ds(s * RPT, RPT)], out.at[c, pl.ds(s * RPT, RPT)])


def _make_spmm(d):
    nbuf = 2 if d == D_HID else 4
    @functools.partial(
        pl.kernel,
        out_type=jax.ShapeDtypeStruct((NC, NP, d), jnp.float32),
        mesh=_sc_mesh(),
        scratch_types=[
            pltpu.VMEM_SHARED((NP, d), jnp.float32),
            pltpu.VMEM_SHARED((NP, d), jnp.float32),
            pltpu.VMEM((CPW, EC), jnp.int32),
            pltpu.VMEM((CPW, EC), jnp.int32),
            [pltpu.VMEM((EC, d), jnp.float32)] * nbuf,
            [pltpu.SemaphoreType.DMA] * nbuf,
        ],
        compiler_params=pltpu.CompilerParams(use_tc_tiling_on_sc=False),
        name=f"spmm{d}",
    )
    def spmm(table, src2d, dst2d, zeros_hbm, out, acc, table_s,
             src_v, dst_v, rows, gsem):
        c = lax.axis_index("c")
        s = lax.axis_index("s")
        wid = s * NC + c
        pltpu.sync_copy(src2d.at[pl.ds(wid * CPW, CPW)], src_v)
        pltpu.sync_copy(dst2d.at[pl.ds(wid * CPW, CPW)], dst_v)
        pltpu.sync_copy(zeros_hbm, acc.at[pl.ds(s * RPT, RPT)])
        pltpu.sync_copy(table.at[pl.ds(s * RPT, RPT)], table_s.at[pl.ds(s * RPT, RPT)])
        plsc.subcore_barrier()

        pltpu.sync_copy(table_s.at[src_v.at[0]], rows[0])
        pltpu.async_copy(rows[0], acc.at[dst_v.at[0]], gsem[0], add=True)
        pltpu.sync_copy(table_s.at[src_v.at[1]], rows[1])
        pltpu.async_copy(rows[1], acc.at[dst_v.at[1]], gsem[1], add=True)

        def body(k, carry):
            for b in range(2):
                j = 2 * k + b
                pltpu.make_async_copy(rows[b], acc.at[dst_v.at[j - 2]], gsem[b]).wait()
                pltpu.sync_copy(table_s.at[src_v.at[j]], rows[b])
                pltpu.async_copy(rows[b], acc.at[dst_v.at[j]], gsem[b], add=True)
            return carry

        lax.fori_loop(1, CPW // 2, body, 0)
        for b in range(2):
            pltpu.make_async_copy(rows[b], acc.at[dst_v.at[CPW - 2 + b]], gsem[b]).wait()
        plsc.subcore_barrier()
        pltpu.sync_copy(acc.at[pl.ds(s * RPT, RPT)], out.at[c, pl.ds(s * RPT, RPT)])

    return spmm


_spmm64 = _make_spmm(D_HID)
_spmm16 = _make_spmm(D_O)


_BN = 2048
_GRID = NP // _BN


def _dis(deg_ref):
    deg = deg_ref[0, :, 0] + deg_ref[1, :, 0] + 1.0
    return lax.rsqrt(deg)


def _tc1a_body(x_ref, w_ref, o_ref):
    o_ref[:] = jnp.dot(x_ref[:], w_ref[:], preferred_element_type=jnp.float32)


def _tc1b_body(h_ref, deg_ref, o_ref):
    o_ref[:] = h_ref[:] * _dis(deg_ref)[:, None]


def _tc2_body(a_ref, h1_ref, deg_ref, w2_ref, b1_ref, o_ref):
    dis = _dis(deg_ref)
    z = dis[:, None] * (a_ref[0] + a_ref[1] + h1_ref[:]) + b1_ref[:]
    z = jnp.maximum(z, 0.0)
    h2 = jnp.dot(z, w2_ref[:], preferred_element_type=jnp.float32)
    o_ref[:] = h2 * dis[:, None]


def _tc3_body(a_ref, h2_ref, deg_ref, b2_ref, o_ref):
    dis = _dis(deg_ref)
    full = dis[:, None] * (a_ref[0] + a_ref[1] + h2_ref[:]) + b2_ref[:]
    o_ref[:] = full[:, :2]


def _row_spec(d):
    return pl.BlockSpec((_BN, d), lambda i: (i, 0))


def _pair_spec(d):
    return pl.BlockSpec((NC, _BN, d), lambda i: (0, i, 0))


def _full_spec(shape):
    return pl.BlockSpec(shape, lambda i: tuple(0 for _ in shape))


def _tc1a(xp, W1):
    return pl.pallas_call(
        _tc1a_body,
        grid=(_GRID,),
        in_specs=[_row_spec(D_IN), _full_spec((D_IN, D_HID))],
        out_specs=_row_spec(D_HID),
        out_shape=jax.ShapeDtypeStruct((NP, D_HID), jnp.float32),
    )(xp, W1)


def _tc1b(h1, deg_pair):
    return pl.pallas_call(
        _tc1b_body,
        grid=(_GRID,),
        in_specs=[_row_spec(D_HID), _pair_spec(8)],
        out_specs=_row_spec(D_HID),
        out_shape=jax.ShapeDtypeStruct((NP, D_HID), jnp.float32),
    )(h1, deg_pair)


def _tc2(acc1, h1p, deg_pair, W2p, b1):
    return pl.pallas_call(
        _tc2_body,
        grid=(_GRID,),
        in_specs=[
            _pair_spec(D_HID), _row_spec(D_HID), _pair_spec(8),
            _full_spec((D_HID, D_O)), _full_spec((1, D_HID)),
        ],
        out_specs=_row_spec(D_O),
        out_shape=jax.ShapeDtypeStruct((NP, D_O), jnp.float32),
    )(acc1, h1p, deg_pair, W2p, b1)


_BO = 2000


def _tc3(acc2, h2p, deg_pair, b2p):
    return pl.pallas_call(
        _tc3_body,
        grid=(N_NODES // _BO,),
        in_specs=[
            pl.BlockSpec((NC, _BO, D_O), lambda i: (0, i, 0)),
            pl.BlockSpec((_BO, D_O), lambda i: (i, 0)),
            pl.BlockSpec((NC, _BO, 8), lambda i: (0, i, 0)),
            _full_spec((1, D_O)),
        ],
        out_specs=pl.BlockSpec((_BO, 2), lambda i: (i, 0)),
        out_shape=jax.ShapeDtypeStruct((N_NODES, 2), jnp.float32),
    )(acc2, h2p, deg_pair, b2p)


def kernel(x, edge_index, W1, b1, W2, b2):
    src = jnp.asarray(edge_index[0], jnp.int32)
    dst = jnp.asarray(edge_index[1], jnp.int32)
    src2d = jnp.concatenate([src, _SRC_PAD]).reshape(NROWS, EC)
    dst2d = jnp.concatenate([dst, _DST_PAD]).reshape(NROWS, EC)
    xp = jnp.pad(x, ((0, NP - N_NODES), (0, 0)))
    W2p = jnp.pad(W2, ((0, 0), (0, D_O - W2.shape[1])))
    b2p = jnp.pad(b2, (0, D_O - b2.shape[0])).reshape(1, D_O)
    b1r = b1.reshape(1, D_HID)

    z8 = jnp.zeros((RPT, 8), jnp.float32)
    z64 = jnp.zeros((RPT, D_HID), jnp.float32)
    z16 = jnp.zeros((RPT, D_O), jnp.float32)
    ones8 = jnp.ones((EC, 8), jnp.float32)

    deg_pair = _deg_kernel(dst2d, z8, ones8)
    h1 = _tc1a(xp, W1)
    h1p = _tc1b(h1, deg_pair)
    acc1 = _spmm64(h1p, src2d, dst2d, z64)
    h2p = _tc2(acc1, h1p, deg_pair, W2p, b1r)
    acc2 = _spmm16(h2p, src2d, dst2d, z16)
    return _tc3(acc2, h2p, deg_pair, b2p)


# speedup vs baseline: 1.3333x; 1.0027x over previous
import functools

import jax
import jax.numpy as jnp
import numpy as np
from jax import lax
from jax.experimental import pallas as pl
from jax.experimental.pallas import tpu as pltpu
from jax.experimental.pallas import tpu_sc as plsc

N_NODES = 10000
N_EDGES = 320000
NP = 10240
EP = 327680
EC = 128
NROWS = EP // EC
NC, NS = 2, 16
NW = NC * NS
CPW = NROWS // NW
RPT = NP // NS

D_IN = 128
D_HID = 64
D_O = 16


def _sc_mesh():
    return plsc.VectorSubcoreMesh(core_axis_name="c", subcore_axis_name="s")


@functools.partial(
    pl.kernel,
    out_type=jax.ShapeDtypeStruct((NC, NP, 8), jnp.float32),
    mesh=_sc_mesh(),
    scratch_types=[
        pltpu.VMEM_SHARED((NP, 8), jnp.float32),
        pltpu.VMEM((CPW, EC), jnp.int32),
        pltpu.VMEM((EC, 8), jnp.float32),
    ],
    compiler_params=pltpu.CompilerParams(use_tc_tiling_on_sc=False),
    name="deg_hist",
)
def _deg_kernel(dst2d, zeros_hbm, ones_hbm, out, acc, idx_v, ones_v):
    c = lax.axis_index("c")
    s = lax.axis_index("s")
    wid = s * NC + c
    pltpu.sync_copy(ones_hbm, ones_v)
    pltpu.sync_copy(dst2d.at[pl.ds(wid * CPW, CPW)], idx_v)
    pltpu.sync_copy(zeros_hbm, acc.at[pl.ds(s * RPT, RPT)])
    plsc.subcore_barrier()

    def body(j, carry):
        pltpu.sync_copy(ones_v, acc.at[idx_v.at[j]], add=True)
        return carry

    lax.fori_loop(0, CPW, body, 0)
    plsc.subcore_barrier()
    pltpu.sync_copy(acc.at[pl.ds(s * RPT, RPT)], out.at[c, pl.ds(s * RPT, RPT)])


def _make_spmm(d):
    nbuf = 2 if d == D_HID else 4
    @functools.partial(
        pl.kernel,
        out_type=jax.ShapeDtypeStruct((NC, NP, d), jnp.float32),
        mesh=_sc_mesh(),
        scratch_types=[
            pltpu.VMEM_SHARED((NP, d), jnp.float32),
            pltpu.VMEM_SHARED((NP, d), jnp.float32),
            pltpu.VMEM((CPW, EC), jnp.int32),
            pltpu.VMEM((CPW, EC), jnp.int32),
            [pltpu.VMEM((EC, d), jnp.float32)] * nbuf,
            [pltpu.SemaphoreType.DMA] * nbuf,
        ],
        compiler_params=pltpu.CompilerParams(use_tc_tiling_on_sc=False),
        name=f"spmm{d}",
    )
    def spmm(table, src2d, dst2d, zeros_hbm, out, acc, table_s,
             src_v, dst_v, rows, gsem):
        c = lax.axis_index("c")
        s = lax.axis_index("s")
        wid = s * NC + c
        pltpu.sync_copy(src2d.at[pl.ds(wid * CPW, CPW)], src_v)
        pltpu.sync_copy(dst2d.at[pl.ds(wid * CPW, CPW)], dst_v)
        pltpu.sync_copy(zeros_hbm, acc.at[pl.ds(s * RPT, RPT)])
        pltpu.sync_copy(table.at[pl.ds(s * RPT, RPT)], table_s.at[pl.ds(s * RPT, RPT)])
        plsc.subcore_barrier()

        pltpu.sync_copy(table_s.at[src_v.at[0]], rows[0])
        pltpu.async_copy(rows[0], acc.at[dst_v.at[0]], gsem[0], add=True)
        pltpu.sync_copy(table_s.at[src_v.at[1]], rows[1])
        pltpu.async_copy(rows[1], acc.at[dst_v.at[1]], gsem[1], add=True)

        def body(k, carry):
            for b in range(2):
                j = 2 * k + b
                pltpu.make_async_copy(rows[b], acc.at[dst_v.at[j - 2]], gsem[b]).wait()
                pltpu.sync_copy(table_s.at[src_v.at[j]], rows[b])
                pltpu.async_copy(rows[b], acc.at[dst_v.at[j]], gsem[b], add=True)
            return carry

        lax.fori_loop(1, CPW // 2, body, 0)
        for b in range(2):
            pltpu.make_async_copy(rows[b], acc.at[dst_v.at[CPW - 2 + b]], gsem[b]).wait()
        plsc.subcore_barrier()
        pltpu.sync_copy(acc.at[pl.ds(s * RPT, RPT)], out.at[c, pl.ds(s * RPT, RPT)])

    return spmm


_spmm64 = _make_spmm(D_HID)
_spmm16 = _make_spmm(D_O)


_BN = 2048
_GRID = NP // _BN


def _dis(deg_ref):
    deg = deg_ref[0, :, 0] + deg_ref[1, :, 0] + 1.0
    return lax.rsqrt(deg)


def _tc1a_body(x_ref, w_ref, o_ref):
    o_ref[:] = jnp.dot(x_ref[:], w_ref[:], preferred_element_type=jnp.float32)


def _tc1b_body(h_ref, deg_ref, o_ref):
    o_ref[:] = h_ref[:] * _dis(deg_ref)[:, None]


def _tc2_body(a_ref, h1_ref, deg_ref, w2_ref, b1_ref, o_ref):
    dis = _dis(deg_ref)
    z = dis[:, None] * (a_ref[0] + a_ref[1] + h1_ref[:]) + b1_ref[:]
    z = jnp.maximum(z, 0.0)
    h2 = jnp.dot(z, w2_ref[:], preferred_element_type=jnp.float32)
    o_ref[:] = h2 * dis[:, None]


def _tc3_body(a_ref, h2_ref, deg_ref, b2_ref, o_ref):
    dis = _dis(deg_ref)
    full = dis[:, None] * (a_ref[0] + a_ref[1] + h2_ref[:]) + b2_ref[:]
    o_ref[:] = full[:, :2]


def _row_spec(d):
    return pl.BlockSpec((_BN, d), lambda i: (i, 0))


def _pair_spec(d):
    return pl.BlockSpec((NC, _BN, d), lambda i: (0, i, 0))


def _full_spec(shape):
    return pl.BlockSpec(shape, lambda i: tuple(0 for _ in shape))


def _tc1a(xp, W1):
    return pl.pallas_call(
        _tc1a_body,
        grid=(_GRID,),
        in_specs=[_row_spec(D_IN), _full_spec((D_IN, D_HID))],
        out_specs=_row_spec(D_HID),
        out_shape=jax.ShapeDtypeStruct((NP, D_HID), jnp.float32),
    )(xp, W1)


def _tc1b(h1, deg_pair):
    return pl.pallas_call(
        _tc1b_body,
        grid=(_GRID,),
        in_specs=[_row_spec(D_HID), _pair_spec(8)],
        out_specs=_row_spec(D_HID),
        out_shape=jax.ShapeDtypeStruct((NP, D_HID), jnp.float32),
    )(h1, deg_pair)


def _tc2(acc1, h1p, deg_pair, W2p, b1):
    return pl.pallas_call(
        _tc2_body,
        grid=(_GRID,),
        in_specs=[
            _pair_spec(D_HID), _row_spec(D_HID), _pair_spec(8),
            _full_spec((D_HID, D_O)), _full_spec((1, D_HID)),
        ],
        out_specs=_row_spec(D_O),
        out_shape=jax.ShapeDtypeStruct((NP, D_O), jnp.float32),
    )(acc1, h1p, deg_pair, W2p, b1)


_BO = 2000


def _tc3(acc2, h2p, deg_pair, b2p):
    return pl.pallas_call(
        _tc3_body,
        grid=(N_NODES // _BO,),
        in_specs=[
            pl.BlockSpec((NC, _BO, D_O), lambda i: (0, i, 0)),
            pl.BlockSpec((_BO, D_O), lambda i: (i, 0)),
            pl.BlockSpec((NC, _BO, 8), lambda i: (0, i, 0)),
            _full_spec((1, D_O)),
        ],
        out_specs=pl.BlockSpec((_BO, 2), lambda i: (i, 0)),
        out_shape=jax.ShapeDtypeStruct((N_NODES, 2), jnp.float32),
    )(acc2, h2p, deg_pair, b2p)


def kernel(x, edge_index, W1, b1, W2, b2):
    src = jnp.asarray(edge_index[0], jnp.int32)
    dst = jnp.asarray(edge_index[1], jnp.int32)
    src2d = jnp.pad(src.reshape(-1, EC), ((0, NROWS - N_EDGES // EC), (0, 0)))
    dst2d = jnp.pad(dst.reshape(-1, EC), ((0, NROWS - N_EDGES // EC), (0, 0)),
                    constant_values=N_NODES)
    xp = jnp.pad(x, ((0, NP - N_NODES), (0, 0)))
    W2p = jnp.pad(W2, ((0, 0), (0, D_O - W2.shape[1])))
    b2p = jnp.pad(b2, (0, D_O - b2.shape[0])).reshape(1, D_O)
    b1r = b1.reshape(1, D_HID)

    z8 = jnp.zeros((RPT, 8), jnp.float32)
    z64 = jnp.zeros((RPT, D_HID), jnp.float32)
    z16 = jnp.zeros((RPT, D_O), jnp.float32)
    ones8 = jnp.ones((EC, 8), jnp.float32)

    deg_pair = _deg_kernel(dst2d, z8, ones8)
    h1 = _tc1a(xp, W1)
    h1p = _tc1b(h1, deg_pair)
    acc1 = _spmm64(h1p, src2d, dst2d, z64)
    h2p = _tc2(acc1, h1p, deg_pair, W2p, b1r)
    acc2 = _spmm16(h2p, src2d, dst2d, z16)
    return _tc3(acc2, h2p, deg_pair, b2p)


# speedup vs baseline: 1.3584x; 1.0188x over previous
import functools

import jax
import jax.numpy as jnp
import numpy as np
from jax import lax
from jax.experimental import pallas as pl
from jax.experimental.pallas import tpu as pltpu
from jax.experimental.pallas import tpu_sc as plsc

N_NODES = 10000
N_EDGES = 320000
NP = 10240
EP = 327680
EC = 128
NROWS = EP // EC
NC, NS = 2, 16
NW = NC * NS
CPW = NROWS // NW
RPT = NP // NS

D_IN = 128
D_HID = 64
D_O = 8


def _sc_mesh():
    return plsc.VectorSubcoreMesh(core_axis_name="c", subcore_axis_name="s")


@functools.partial(
    pl.kernel,
    out_type=jax.ShapeDtypeStruct((NC, NP, 8), jnp.float32),
    mesh=_sc_mesh(),
    scratch_types=[
        pltpu.VMEM_SHARED((NP, 8), jnp.float32),
        pltpu.VMEM((CPW, EC), jnp.int32),
        pltpu.VMEM((EC, 8), jnp.float32),
        pltpu.SemaphoreType.DMA,
        pltpu.SemaphoreType.DMA,
    ],
    compiler_params=pltpu.CompilerParams(use_tc_tiling_on_sc=False),
    name="deg_hist",
)
def _deg_kernel(dst2d, zeros_hbm, ones_hbm, out, acc, idx_v, ones_v, sem0, sem1):
    c = lax.axis_index("c")
    s = lax.axis_index("s")
    wid = s * NC + c
    pltpu.sync_copy(ones_hbm, ones_v)
    pltpu.sync_copy(dst2d.at[pl.ds(wid * CPW, CPW)], idx_v)
    pltpu.sync_copy(zeros_hbm, acc.at[pl.ds(s * RPT, RPT)])
    plsc.subcore_barrier()

    pltpu.async_copy(ones_v, acc.at[idx_v.at[0]], sem0, add=True)
    pltpu.async_copy(ones_v, acc.at[idx_v.at[1]], sem1, add=True)

    def body(k, carry):
        pltpu.make_async_copy(ones_v, acc.at[idx_v.at[2 * k - 2]], sem0).wait()
        pltpu.async_copy(ones_v, acc.at[idx_v.at[2 * k]], sem0, add=True)
        pltpu.make_async_copy(ones_v, acc.at[idx_v.at[2 * k - 1]], sem1).wait()
        pltpu.async_copy(ones_v, acc.at[idx_v.at[2 * k + 1]], sem1, add=True)
        return carry

    lax.fori_loop(1, CPW // 2, body, 0)
    pltpu.make_async_copy(ones_v, acc.at[idx_v.at[CPW - 2]], sem0).wait()
    pltpu.make_async_copy(ones_v, acc.at[idx_v.at[CPW - 1]], sem1).wait()
    plsc.subcore_barrier()
    pltpu.sync_copy(acc.at[pl.ds(s * RPT, RPT)], out.at[c, pl.ds(s * RPT, RPT)])


def _make_spmm(d):
    nbuf = 2 if d == D_HID else 4
    @functools.partial(
        pl.kernel,
        out_type=jax.ShapeDtypeStruct((NC, NP, d), jnp.float32),
        mesh=_sc_mesh(),
        scratch_types=[
            pltpu.VMEM_SHARED((NP, d), jnp.float32),
            pltpu.VMEM_SHARED((NP, d), jnp.float32),
            pltpu.VMEM((CPW, EC), jnp.int32),
            pltpu.VMEM((CPW, EC), jnp.int32),
            [pltpu.VMEM((EC, d), jnp.float32)] * nbuf,
            [pltpu.SemaphoreType.DMA] * nbuf,
        ],
        compiler_params=pltpu.CompilerParams(use_tc_tiling_on_sc=False),
        name=f"spmm{d}",
    )
    def spmm(table, src2d, dst2d, zeros_hbm, out, acc, table_s,
             src_v, dst_v, rows, gsem):
        c = lax.axis_index("c")
        s = lax.axis_index("s")
        wid = s * NC + c
        pltpu.sync_copy(src2d.at[pl.ds(wid * CPW, CPW)], src_v)
        pltpu.sync_copy(dst2d.at[pl.ds(wid * CPW, CPW)], dst_v)
        pltpu.sync_copy(zeros_hbm, acc.at[pl.ds(s * RPT, RPT)])
        pltpu.sync_copy(table.at[pl.ds(s * RPT, RPT)], table_s.at[pl.ds(s * RPT, RPT)])
        plsc.subcore_barrier()

        pltpu.sync_copy(table_s.at[src_v.at[0]], rows[0])
        pltpu.async_copy(rows[0], acc.at[dst_v.at[0]], gsem[0], add=True)
        pltpu.sync_copy(table_s.at[src_v.at[1]], rows[1])
        pltpu.async_copy(rows[1], acc.at[dst_v.at[1]], gsem[1], add=True)

        def body(k, carry):
            for b in range(2):
                j = 2 * k + b
                pltpu.make_async_copy(rows[b], acc.at[dst_v.at[j - 2]], gsem[b]).wait()
                pltpu.sync_copy(table_s.at[src_v.at[j]], rows[b])
                pltpu.async_copy(rows[b], acc.at[dst_v.at[j]], gsem[b], add=True)
            return carry

        lax.fori_loop(1, CPW // 2, body, 0)
        for b in range(2):
            pltpu.make_async_copy(rows[b], acc.at[dst_v.at[CPW - 2 + b]], gsem[b]).wait()
        plsc.subcore_barrier()
        pltpu.sync_copy(acc.at[pl.ds(s * RPT, RPT)], out.at[c, pl.ds(s * RPT, RPT)])

    return spmm


_spmm64 = _make_spmm(D_HID)
_spmm16 = _make_spmm(D_O)


_BN = 2048
_GRID = NP // _BN


def _dis(deg_ref):
    deg = deg_ref[0, :, 0] + deg_ref[1, :, 0] + 1.0
    return lax.rsqrt(deg)


def _tc1a_body(x_ref, w_ref, o_ref):
    o_ref[:] = jnp.dot(x_ref[:], w_ref[:], preferred_element_type=jnp.float32)


def _tc1b_body(h_ref, deg_ref, o_ref):
    o_ref[:] = h_ref[:] * _dis(deg_ref)[:, None]


def _tc2_body(a_ref, h1_ref, deg_ref, w2_ref, b1_ref, o_ref):
    dis = _dis(deg_ref)
    z = dis[:, None] * (a_ref[0] + a_ref[1] + h1_ref[:]) + b1_ref[:]
    z = jnp.maximum(z, 0.0)
    h2 = jnp.dot(z, w2_ref[:], preferred_element_type=jnp.float32)
    o_ref[:] = h2 * dis[:, None]


def _tc3_body(a_ref, h2_ref, deg_ref, b2_ref, o_ref):
    dis = _dis(deg_ref)
    full = dis[:, None] * (a_ref[0] + a_ref[1] + h2_ref[:]) + b2_ref[:]
    o_ref[:] = full[:, :2]


def _row_spec(d):
    return pl.BlockSpec((_BN, d), lambda i: (i, 0))


def _pair_spec(d):
    return pl.BlockSpec((NC, _BN, d), lambda i: (0, i, 0))


def _full_spec(shape):
    return pl.BlockSpec(shape, lambda i: tuple(0 for _ in shape))


def _tc1a(xp, W1):
    return pl.pallas_call(
        _tc1a_body,
        grid=(_GRID,),
        in_specs=[_row_spec(D_IN), _full_spec((D_IN, D_HID))],
        out_specs=_row_spec(D_HID),
        out_shape=jax.ShapeDtypeStruct((NP, D_HID), jnp.float32),
    )(xp, W1)


def _tc1b(h1, deg_pair):
    return pl.pallas_call(
        _tc1b_body,
        grid=(_GRID,),
        in_specs=[_row_spec(D_HID), _pair_spec(8)],
        out_specs=_row_spec(D_HID),
        out_shape=jax.ShapeDtypeStruct((NP, D_HID), jnp.float32),
    )(h1, deg_pair)


def _tc2(acc1, h1p, deg_pair, W2p, b1):
    return pl.pallas_call(
        _tc2_body,
        grid=(_GRID,),
        in_specs=[
            _pair_spec(D_HID), _row_spec(D_HID), _pair_spec(8),
            _full_spec((D_HID, D_O)), _full_spec((1, D_HID)),
        ],
        out_specs=_row_spec(D_O),
        out_shape=jax.ShapeDtypeStruct((NP, D_O), jnp.float32),
    )(acc1, h1p, deg_pair, W2p, b1)


_BO = 2000


def _tc3(acc2, h2p, deg_pair, b2p):
    return pl.pallas_call(
        _tc3_body,
        grid=(N_NODES // _BO,),
        in_specs=[
            pl.BlockSpec((NC, _BO, D_O), lambda i: (0, i, 0)),
            pl.BlockSpec((_BO, D_O), lambda i: (i, 0)),
            pl.BlockSpec((NC, _BO, 8), lambda i: (0, i, 0)),
            _full_spec((1, D_O)),
        ],
        out_specs=pl.BlockSpec((_BO, 2), lambda i: (i, 0)),
        out_shape=jax.ShapeDtypeStruct((N_NODES, 2), jnp.float32),
    )(acc2, h2p, deg_pair, b2p)


def kernel(x, edge_index, W1, b1, W2, b2):
    src = jnp.asarray(edge_index[0], jnp.int32)
    dst = jnp.asarray(edge_index[1], jnp.int32)
    src2d = jnp.pad(src.reshape(-1, EC), ((0, NROWS - N_EDGES // EC), (0, 0)))
    dst2d = jnp.pad(dst.reshape(-1, EC), ((0, NROWS - N_EDGES // EC), (0, 0)),
                    constant_values=N_NODES)
    xp = jnp.pad(x, ((0, NP - N_NODES), (0, 0)))
    W2p = jnp.pad(W2, ((0, 0), (0, D_O - W2.shape[1])))
    b2p = jnp.pad(b2, (0, D_O - b2.shape[0])).reshape(1, D_O)
    b1r = b1.reshape(1, D_HID)

    z8 = jnp.zeros((RPT, 8), jnp.float32)
    z64 = jnp.zeros((RPT, D_HID), jnp.float32)
    z16 = jnp.zeros((RPT, D_O), jnp.float32)
    ones8 = jnp.ones((EC, 8), jnp.float32)

    deg_pair = _deg_kernel(dst2d, z8, ones8)
    h1 = _tc1a(xp, W1)
    h1p = _tc1b(h1, deg_pair)
    acc1 = _spmm64(h1p, src2d, dst2d, z64)
    h2p = _tc2(acc1, h1p, deg_pair, W2p, b1r)
    acc2 = _spmm16(h2p, src2d, dst2d, z16)
    return _tc3(acc2, h2p, deg_pair, b2p)


# speedup vs baseline: 1.3930x; 1.0255x over previous
import functools

import jax
import jax.numpy as jnp
import numpy as np
from jax import lax
from jax.experimental import pallas as pl
from jax.experimental.pallas import tpu as pltpu
from jax.experimental.pallas import tpu_sc as plsc

N_NODES = 10000
N_EDGES = 320000
NP = 10240
EP = 327680
EC = 128
NROWS = EP // EC
NC, NS = 2, 16
NW = NC * NS
CPW = NROWS // NW
RPT = NP // NS

D_IN = 128
D_HID = 64
D_O = 8


def _sc_mesh():
    return plsc.VectorSubcoreMesh(core_axis_name="c", subcore_axis_name="s")


@functools.partial(
    pl.kernel,
    out_type=jax.ShapeDtypeStruct((NC, NP, 8), jnp.float32),
    mesh=_sc_mesh(),
    scratch_types=[
        pltpu.VMEM_SHARED((NP, 8), jnp.float32),
        pltpu.VMEM((CPW, EC), jnp.int32),
        pltpu.VMEM((EC, 8), jnp.float32),
        pltpu.SemaphoreType.DMA,
        pltpu.SemaphoreType.DMA,
        pltpu.SemaphoreType.DMA,
    ],
    compiler_params=pltpu.CompilerParams(use_tc_tiling_on_sc=False),
    name="deg_hist",
)
def _deg_kernel(dst2d, zeros_hbm, ones_hbm, out, acc, idx_v, ones_v, sem0, sem1, sem2):
    c = lax.axis_index("c")
    s = lax.axis_index("s")
    wid = s * NC + c
    cp1 = pltpu.async_copy(ones_hbm, ones_v, sem0)
    cp2 = pltpu.async_copy(dst2d.at[pl.ds(wid * CPW, CPW)], idx_v, sem1)
    cp3 = pltpu.async_copy(zeros_hbm, acc.at[pl.ds(s * RPT, RPT)], sem2)
    cp1.wait()
    cp2.wait()
    cp3.wait()
    plsc.subcore_barrier()

    pltpu.async_copy(ones_v, acc.at[idx_v.at[0]], sem0, add=True)
    pltpu.async_copy(ones_v, acc.at[idx_v.at[1]], sem1, add=True)

    def body(k, carry):
        pltpu.make_async_copy(ones_v, acc.at[idx_v.at[2 * k - 2]], sem0).wait()
        pltpu.async_copy(ones_v, acc.at[idx_v.at[2 * k]], sem0, add=True)
        pltpu.make_async_copy(ones_v, acc.at[idx_v.at[2 * k - 1]], sem1).wait()
        pltpu.async_copy(ones_v, acc.at[idx_v.at[2 * k + 1]], sem1, add=True)
        return carry

    lax.fori_loop(1, CPW // 2, body, 0)
    pltpu.make_async_copy(ones_v, acc.at[idx_v.at[CPW - 2]], sem0).wait()
    pltpu.make_async_copy(ones_v, acc.at[idx_v.at[CPW - 1]], sem1).wait()
    plsc.subcore_barrier()
    pltpu.sync_copy(acc.at[pl.ds(s * RPT, RPT)], out.at[c, pl.ds(s * RPT, RPT)])


def _make_spmm(d):
    nbuf = 2 if d == D_HID else 4
    @functools.partial(
        pl.kernel,
        out_type=jax.ShapeDtypeStruct((NC, NP, d), jnp.float32),
        mesh=_sc_mesh(),
        scratch_types=[
            pltpu.VMEM_SHARED((NP, d), jnp.float32),
            pltpu.VMEM_SHARED((NP, d), jnp.float32),
            pltpu.VMEM((CPW, EC), jnp.int32),
            pltpu.VMEM((CPW, EC), jnp.int32),
            [pltpu.VMEM((EC, d), jnp.float32)] * nbuf,
            [pltpu.SemaphoreType.DMA] * nbuf,
            pltpu.SemaphoreType.DMA,
            pltpu.SemaphoreType.DMA,
        ],
        compiler_params=pltpu.CompilerParams(use_tc_tiling_on_sc=False),
        name=f"spmm{d}",
    )
    def spmm(table, src2d, dst2d, zeros_hbm, out, acc, table_s,
             src_v, dst_v, rows, gsem, psem0, psem1):
        c = lax.axis_index("c")
        s = lax.axis_index("s")
        wid = s * NC + c
        cp1 = pltpu.async_copy(src2d.at[pl.ds(wid * CPW, CPW)], src_v, gsem[0])
        cp2 = pltpu.async_copy(dst2d.at[pl.ds(wid * CPW, CPW)], dst_v, gsem[1])
        cp3 = pltpu.async_copy(zeros_hbm, acc.at[pl.ds(s * RPT, RPT)], psem0)
        cp4 = pltpu.async_copy(
            table.at[pl.ds(s * RPT, RPT)], table_s.at[pl.ds(s * RPT, RPT)], psem1)
        cp1.wait()
        cp2.wait()
        cp3.wait()
        cp4.wait()
        plsc.subcore_barrier()

        pltpu.sync_copy(table_s.at[src_v.at[0]], rows[0])
        pltpu.async_copy(rows[0], acc.at[dst_v.at[0]], gsem[0], add=True)
        pltpu.sync_copy(table_s.at[src_v.at[1]], rows[1])
        pltpu.async_copy(rows[1], acc.at[dst_v.at[1]], gsem[1], add=True)

        def body(k, carry):
            for b in range(2):
                j = 2 * k + b
                pltpu.make_async_copy(rows[b], acc.at[dst_v.at[j - 2]], gsem[b]).wait()
                pltpu.sync_copy(table_s.at[src_v.at[j]], rows[b])
                pltpu.async_copy(rows[b], acc.at[dst_v.at[j]], gsem[b], add=True)
            return carry

        lax.fori_loop(1, CPW // 2, body, 0)
        for b in range(2):
            pltpu.make_async_copy(rows[b], acc.at[dst_v.at[CPW - 2 + b]], gsem[b]).wait()
        plsc.subcore_barrier()
        pltpu.sync_copy(acc.at[pl.ds(s * RPT, RPT)], out.at[c, pl.ds(s * RPT, RPT)])

    return spmm


_spmm64 = _make_spmm(D_HID)
_spmm16 = _make_spmm(D_O)


_BN = 2048
_GRID = NP // _BN


def _dis(deg_ref):
    deg = deg_ref[0, :, 0] + deg_ref[1, :, 0] + 1.0
    return lax.rsqrt(deg)


def _tc1a_body(x_ref, w_ref, o_ref):
    o_ref[:] = jnp.dot(x_ref[:], w_ref[:], preferred_element_type=jnp.float32)


def _tc1b_body(h_ref, deg_ref, o_ref):
    o_ref[:] = h_ref[:] * _dis(deg_ref)[:, None]


def _tc2_body(a_ref, h1_ref, deg_ref, w2_ref, b1_ref, o_ref):
    dis = _dis(deg_ref)
    z = dis[:, None] * (a_ref[0] + a_ref[1] + h1_ref[:]) + b1_ref[:]
    z = jnp.maximum(z, 0.0)
    h2 = jnp.dot(z, w2_ref[:], preferred_element_type=jnp.float32)
    o_ref[:] = h2 * dis[:, None]


def _tc3_body(a_ref, h2_ref, deg_ref, b2_ref, o_ref):
    dis = _dis(deg_ref)
    full = dis[:, None] * (a_ref[0] + a_ref[1] + h2_ref[:]) + b2_ref[:]
    o_ref[:] = full[:, :2]


def _row_spec(d):
    return pl.BlockSpec((_BN, d), lambda i: (i, 0))


def _pair_spec(d):
    return pl.BlockSpec((NC, _BN, d), lambda i: (0, i, 0))


def _full_spec(shape):
    return pl.BlockSpec(shape, lambda i: tuple(0 for _ in shape))


def _tc1a(xp, W1):
    return pl.pallas_call(
        _tc1a_body,
        grid=(_GRID,),
        in_specs=[_row_spec(D_IN), _full_spec((D_IN, D_HID))],
        out_specs=_row_spec(D_HID),
        out_shape=jax.ShapeDtypeStruct((NP, D_HID), jnp.float32),
    )(xp, W1)


def _tc1b(h1, deg_pair):
    return pl.pallas_call(
        _tc1b_body,
        grid=(_GRID,),
        in_specs=[_row_spec(D_HID), _pair_spec(8)],
        out_specs=_row_spec(D_HID),
        out_shape=jax.ShapeDtypeStruct((NP, D_HID), jnp.float32),
    )(h1, deg_pair)


def _tc2(acc1, h1p, deg_pair, W2p, b1):
    return pl.pallas_call(
        _tc2_body,
        grid=(_GRID,),
        in_specs=[
            _pair_spec(D_HID), _row_spec(D_HID), _pair_spec(8),
            _full_spec((D_HID, D_O)), _full_spec((1, D_HID)),
        ],
        out_specs=_row_spec(D_O),
        out_shape=jax.ShapeDtypeStruct((NP, D_O), jnp.float32),
    )(acc1, h1p, deg_pair, W2p, b1)


_BO = 2000


def _tc3(acc2, h2p, deg_pair, b2p):
    return pl.pallas_call(
        _tc3_body,
        grid=(N_NODES // _BO,),
        in_specs=[
            pl.BlockSpec((NC, _BO, D_O), lambda i: (0, i, 0)),
            pl.BlockSpec((_BO, D_O), lambda i: (i, 0)),
            pl.BlockSpec((NC, _BO, 8), lambda i: (0, i, 0)),
            _full_spec((1, D_O)),
        ],
        out_specs=pl.BlockSpec((_BO, 2), lambda i: (i, 0)),
        out_shape=jax.ShapeDtypeStruct((N_NODES, 2), jnp.float32),
    )(acc2, h2p, deg_pair, b2p)


def kernel(x, edge_index, W1, b1, W2, b2):
    src = jnp.asarray(edge_index[0], jnp.int32)
    dst = jnp.asarray(edge_index[1], jnp.int32)
    src2d = jnp.pad(src.reshape(-1, EC), ((0, NROWS - N_EDGES // EC), (0, 0)))
    dst2d = jnp.pad(dst.reshape(-1, EC), ((0, NROWS - N_EDGES // EC), (0, 0)),
                    constant_values=N_NODES)
    xp = jnp.pad(x, ((0, NP - N_NODES), (0, 0)))
    W2p = jnp.pad(W2, ((0, 0), (0, D_O - W2.shape[1])))
    b2p = jnp.pad(b2, (0, D_O - b2.shape[0])).reshape(1, D_O)
    b1r = b1.reshape(1, D_HID)

    z8 = jnp.zeros((RPT, 8), jnp.float32)
    z64 = jnp.zeros((RPT, D_HID), jnp.float32)
    z16 = jnp.zeros((RPT, D_O), jnp.float32)
    ones8 = jnp.ones((EC, 8), jnp.float32)

    deg_pair = _deg_kernel(dst2d, z8, ones8)
    h1 = _tc1a(xp, W1)
    h1p = _tc1b(h1, deg_pair)
    acc1 = _spmm64(h1p, src2d, dst2d, z64)
    h2p = _tc2(acc1, h1p, deg_pair, W2p, b1r)
    acc2 = _spmm16(h2p, src2d, dst2d, z16)
    return _tc3(acc2, h2p, deg_pair, b2p)
